# Initial kernel scaffold; baseline (speedup 1.0000x reference)
import jax
import jax.numpy as jnp
from jax.experimental import pallas as pl


def kernel(images, conv_w, conv_b, fc_w, fc_b):
    raise NotImplementedError("write your pallas kernel here")


# speedup vs baseline: 1.7880x; 1.7880x over previous
import functools

import jax
import jax.numpy as jnp
from jax.experimental import pallas as pl
from jax.experimental.pallas import tpu as pltpu

_VMEM_LIMIT = 32 * 1024 * 1024


def _rup(x, m):
    return ((x + m - 1) // m) * m


def _enc_kernel(xm_ref, q_ref, wk_ref, bc_ref, pm_ref, wf_ref,
                bf_ref, pooled_ref, attrs_ref, gacc_ref, carry_ref,
                *, kh, wo, wop, cin, inv_count):
    oy = pl.program_id(1)
    rows = 2 * kh + 1
    s = pooled_ref.shape[2]
    cpad = pooled_ref.shape[3]

    @pl.when(oy == 0)
    def _():
        gacc_ref[...] = jnp.zeros_like(gacc_ref)
        carry_ref[...] = jnp.zeros_like(carry_ref)

    w = xm_ref.shape[3]
    xt = carry_ref[...].reshape(cin, 1, w)
    xmp = xm_ref[...].reshape(cin, kh, 2, w)
    x_odd = xmp[:, :, 0, :]
    x_even_tail = xmp[:, :, 1, :]
    carry_ref[...] = x_even_tail[:, kh - 1, :]
    x = jnp.concatenate([xt, x_even_tail, x_odd], axis=1)
    x2 = x.reshape(cin * rows, w).astype(jnp.bfloat16)

    z = jnp.dot(x2, q_ref[...], preferred_element_type=jnp.float32)
    z = z.astype(jnp.bfloat16).reshape(cin, rows, 3 * wop)

    dy_slices = (z[:, 0:kh], z[:, kh + 1:2 * kh + 1], z[:, 1:kh + 1])
    pieces = []
    for dy in range(3):
        for dx in range(3):
            pieces.append(dy_slices[dy][:, :, dx * wop:(dx + 1) * wop])
    t = jnp.concatenate(pieces, axis=0).reshape(9 * cin, kh * wop)

    conv = jax.lax.dot_general(t, wk_ref[...], (((0,), (0,)), ((), ())),
                               preferred_element_type=jnp.float32)
    conv = conv.reshape(kh, wop, cpad)
    row_acc = jnp.sum(jnp.maximum(conv + bc_ref[...][None], 0.0),
                      axis=0)[:wo]

    pooled = jnp.dot(pm_ref[...], row_acc, preferred_element_type=jnp.float32)
    pooled_ref[...] = pooled.reshape(1, 1, s, cpad)

    gacc_ref[...] += jnp.sum(row_acc, axis=0, keepdims=True)

    @pl.when(oy == pl.num_programs(1) - 1)
    def _():
        g = gacc_ref[...] * inv_count
        a = jnp.dot(g, wf_ref[...], preferred_element_type=jnp.float32)
        attrs_ref[...] = (a + bf_ref[...]).reshape(1, 1, -1)


@jax.jit
def _forward(images, conv_w, conv_b, fc_w, fc_b):
    b, cin, h, w = images.shape
    cenc = conv_w.shape[0]
    na = fc_w.shape[0]
    s = 14
    ho = (h - 1) // 2 + 1
    wo = (w - 1) // 2 + 1
    kh, kw = ho // s, wo // s
    wop = _rup(wo, 128)
    cpad = _rup(cenc, 128)
    apad = _rup(na, 128)

    wcol = jnp.arange(w)[:, None]
    xs = jnp.arange(wop)[None, :]
    q = jnp.concatenate(
        [((wcol == 2 * xs + dx - 1) & (xs < wo)) for dx in range(3)],
        axis=1).astype(jnp.bfloat16)

    wt = jnp.transpose(conv_w, (2, 3, 1, 0))
    wt = jnp.pad(wt, ((0, 0), (0, 0), (0, 0), (0, cpad - cenc)))
    wk = wt.reshape(9 * cin, cpad).astype(jnp.bfloat16)
    bc = jnp.pad(conv_b, (0, cpad - cenc)).reshape(1, cpad).astype(jnp.float32)

    pm = (jnp.arange(wo)[None, :] // kw == jnp.arange(s)[:, None])
    pm = pm.astype(jnp.float32) / float(kh * kw)

    wf = jnp.pad(jnp.transpose(fc_w),
                 ((0, cpad - cenc), (0, apad - na))).astype(jnp.float32)
    bf2 = jnp.pad(fc_b, (0, apad - na)).reshape(1, apad).astype(jnp.float32)

    cost = pl.CostEstimate(
        flops=(2 * b * ho * wo * 9 * cin * cpad
               + 2 * b * s * cin * (2 * kh + 1) * w * 3 * wop
               + 2 * b * s * s * wo * cpad),
        transcendentals=0,
        bytes_accessed=(b * cin * (h + s) * w * 4
                        + b * s * s * cpad * 4 + b * apad * 4),
    )

    pooled, attrs = pl.pallas_call(
        functools.partial(_enc_kernel, kh=kh, wo=wo, wop=wop, cin=cin,
                          inv_count=1.0 / float(ho * wo)),
        out_shape=(
            jax.ShapeDtypeStruct((b, s, s, cpad), jnp.float32),
            jax.ShapeDtypeStruct((b, 1, apad), jnp.float32),
        ),
        grid=(b, s),
        in_specs=[
            pl.BlockSpec((1, cin, 2 * kh, w), lambda bb, oy: (bb, 0, oy, 0)),
            pl.BlockSpec((w, 3 * wop), lambda bb, oy: (0, 0)),
            pl.BlockSpec((9 * cin, cpad), lambda bb, oy: (0, 0)),
            pl.BlockSpec((1, cpad), lambda bb, oy: (0, 0)),
            pl.BlockSpec((s, wo), lambda bb, oy: (0, 0)),
            pl.BlockSpec((cpad, apad), lambda bb, oy: (0, 0)),
            pl.BlockSpec((1, apad), lambda bb, oy: (0, 0)),
        ],
        out_specs=(
            pl.BlockSpec((1, 1, s, cpad), lambda bb, oy: (bb, oy, 0, 0)),
            pl.BlockSpec((1, 1, apad), lambda bb, oy: (bb, 0, 0)),
        ),
        scratch_shapes=[pltpu.VMEM((1, cpad), jnp.float32),
                        pltpu.VMEM((cin, w), jnp.float32)],
        compiler_params=pltpu.CompilerParams(
            dimension_semantics=("parallel", "arbitrary"),
            vmem_limit_bytes=_VMEM_LIMIT),
        cost_estimate=cost,
    )(images, q, wk, bc, pm, wf, bf2)

    return pooled[:, :, :, :cenc], attrs[:, 0, :na]


def kernel(images, conv_w, conv_b, fc_w, fc_b):
    return _forward(images, conv_w, conv_b, fc_w, fc_b)


# speedup vs baseline: 2.0739x; 1.1599x over previous
import functools

import jax
import jax.numpy as jnp
from jax.experimental import pallas as pl
from jax.experimental.pallas import tpu as pltpu

_VMEM_LIMIT = 32 * 1024 * 1024
_BB = 2


def _rup(x, m):
    return ((x + m - 1) // m) * m


def _enc_kernel(xm_ref, q_ref, wk_ref, bc_ref, pm_ref, wf_ref,
                bf_ref, pooled_ref, attrs_ref, gacc_ref, zc_ref,
                *, kh, wop, cin, bb, inv_count):
    oy = pl.program_id(1)
    s = pooled_ref.shape[2]
    cpad = pooled_ref.shape[3]
    w3 = 3 * wop

    @pl.when(oy == 0)
    def _():
        gacc_ref[...] = jnp.zeros_like(gacc_ref)
        zc_ref[...] = jnp.zeros_like(zc_ref)

    x2 = xm_ref[...].reshape(bb * cin * kh, -1).astype(jnp.bfloat16)

    z = jnp.dot(x2, q_ref[...], preferred_element_type=jnp.float32)
    z = z.astype(jnp.bfloat16)

    bias = bc_ref[...]
    pmx = pm_ref[...]
    wk = wk_ref[...]
    for i in range(bb):
        zi = z[i * cin * kh:(i + 1) * cin * kh, :].reshape(cin, kh, 6 * wop)
        zc = zc_ref[i * cin:(i + 1) * cin, :].reshape(cin, 1, w3)
        dy0 = jnp.concatenate([zc, zi[:, 0:kh - 1, w3:2 * w3]], axis=1)
        zc_ref[i * cin:(i + 1) * cin, :] = zi[:, kh - 1, w3:2 * w3]
        pieces = []
        for dy_block in (dy0, zi[:, :, 0:w3], zi[:, :, w3:2 * w3]):
            for dx in range(3):
                pieces.append(dy_block[:, :, dx * wop:(dx + 1) * wop])
        t = jnp.concatenate(pieces, axis=0)

        conv = jax.lax.dot_general(t, wk, (((0,), (0,)), ((), ())),
                                   preferred_element_type=jnp.float32)

        row_acc = jnp.maximum(conv[0] + bias, 0.0)
        for y in range(1, kh):
            row_acc += jnp.maximum(conv[y] + bias, 0.0)

        sacc = jnp.dot(pmx, row_acc, preferred_element_type=jnp.float32)
        pooled_ref[i, 0] = sacc[:s]
        gacc_ref[i:i + 1, :] += sacc[s:s + 1]

    @pl.when(oy == pl.num_programs(1) - 1)
    def _():
        g = gacc_ref[...] * inv_count
        a = jnp.dot(g, wf_ref[...], preferred_element_type=jnp.float32)
        attrs_ref[...] = (a + bf_ref[...]).reshape(bb, 1, -1)


@jax.jit
def _forward(images, conv_w, conv_b, fc_w, fc_b):
    b, cin, h, w = images.shape
    cenc = conv_w.shape[0]
    na = fc_w.shape[0]
    s = 14
    ho = (h - 1) // 2 + 1
    wo = (w - 1) // 2 + 1
    kh, kw = ho // s, wo // s
    wop = _rup(wo, 128)
    cpad = _rup(cenc, 128)
    apad = _rup(na, 128)
    bb = _BB

    xp = images.reshape(b, cin, h // 2, 2 * w)

    wcol = jnp.arange(w)[:, None]
    xs = jnp.arange(wop)[None, :]
    qhalf = [((wcol == 2 * xs + dx - 1) & (xs < wo)) for dx in range(3)]
    zblk = jnp.zeros((w, wop), jnp.bool_)
    q = jnp.block([[qhalf[0], qhalf[1], qhalf[2], zblk, zblk, zblk],
                   [zblk, zblk, zblk, qhalf[0], qhalf[1], qhalf[2]]])
    q = q.astype(jnp.bfloat16)

    wt = jnp.transpose(conv_w, (2, 3, 1, 0))
    wt = jnp.pad(wt, ((0, 0), (0, 0), (0, 0), (0, cpad - cenc)))
    wk = wt.reshape(9 * cin, cpad).astype(jnp.bfloat16)
    bc = jnp.pad(conv_b, (0, cpad - cenc)).reshape(1, cpad).astype(jnp.float32)

    xcol = jnp.arange(wop)[None, :]
    pm = (xcol // kw == jnp.arange(s)[:, None]) & (xcol < wo)
    pm = pm.astype(jnp.float32) / float(kh * kw)
    pm = jnp.concatenate([pm, (xcol < wo).astype(jnp.float32)], axis=0)

    wf = jnp.pad(jnp.transpose(fc_w),
                 ((0, cpad - cenc), (0, apad - na))).astype(jnp.float32)
    bf2 = jnp.pad(fc_b, (0, apad - na)).reshape(1, apad).astype(jnp.float32)

    cost = pl.CostEstimate(
        flops=(2 * b * ho * wo * 9 * cin * cpad
               + 2 * b * s * cin * kh * 2 * w * 6 * wop
               + 2 * b * s * (s + 1) * wop * cpad),
        transcendentals=0,
        bytes_accessed=(b * cin * h * w * 4
                        + b * s * s * cpad * 4 + b * apad * 4),
    )

    pooled, attrs = pl.pallas_call(
        functools.partial(_enc_kernel, kh=kh, wop=wop, cin=cin, bb=bb,
                          inv_count=1.0 / float(ho * wo)),
        out_shape=(
            jax.ShapeDtypeStruct((b, s, s, cpad), jnp.float32),
            jax.ShapeDtypeStruct((b, 1, apad), jnp.float32),
        ),
        grid=(b // bb, s),
        in_specs=[
            pl.BlockSpec((bb, cin, kh, 2 * w), lambda bb_, oy: (bb_, 0, oy, 0)),
            pl.BlockSpec((2 * w, 6 * wop), lambda bb_, oy: (0, 0)),
            pl.BlockSpec((9 * cin, cpad), lambda bb_, oy: (0, 0)),
            pl.BlockSpec((1, cpad), lambda bb_, oy: (0, 0)),
            pl.BlockSpec((s + 1, wop), lambda bb_, oy: (0, 0)),
            pl.BlockSpec((cpad, apad), lambda bb_, oy: (0, 0)),
            pl.BlockSpec((1, apad), lambda bb_, oy: (0, 0)),
        ],
        out_specs=(
            pl.BlockSpec((bb, 1, s, cpad), lambda bb_, oy: (bb_, oy, 0, 0)),
            pl.BlockSpec((bb, 1, apad), lambda bb_, oy: (bb_, 0, 0)),
        ),
        scratch_shapes=[pltpu.VMEM((bb, cpad), jnp.float32),
                        pltpu.VMEM((bb * cin, 3 * wop), jnp.bfloat16)],
        compiler_params=pltpu.CompilerParams(
            dimension_semantics=("parallel", "arbitrary"),
            vmem_limit_bytes=_VMEM_LIMIT),
        cost_estimate=cost,
    )(xp, q, wk, bc, pm, wf, bf2)

    return pooled[:, :, :, :cenc], attrs[:, 0, :na]


def kernel(images, conv_w, conv_b, fc_w, fc_b):
    return _forward(images, conv_w, conv_b, fc_w, fc_b)


# speedup vs baseline: 2.3756x; 1.1455x over previous
import functools

import jax
import jax.numpy as jnp
from jax.experimental import pallas as pl
from jax.experimental.pallas import tpu as pltpu

_VMEM_LIMIT = 32 * 1024 * 1024
_BB = 4


def _rup(x, m):
    return ((x + m - 1) // m) * m


def _enc_kernel(xm_ref, q_ref, wk_ref, bc_ref, pm_ref, wf_ref,
                bf_ref, pooled_ref, attrs_ref, gacc_ref, zc_ref,
                *, kh, wop, cin, bb, inv_count):
    oy = pl.program_id(1)
    s = pooled_ref.shape[2]
    cpad = pooled_ref.shape[3]
    w3 = 3 * wop

    @pl.when(oy == 0)
    def _():
        gacc_ref[...] = jnp.zeros_like(gacc_ref)
        zc_ref[...] = jnp.zeros_like(zc_ref)

    x2 = xm_ref[...].reshape(bb * cin * kh, -1).astype(jnp.bfloat16)

    z = jnp.dot(x2, q_ref[...], preferred_element_type=jnp.float32)
    z = z.astype(jnp.bfloat16)

    bias = bc_ref[...]
    pmx = pm_ref[...]
    wk = wk_ref[...]
    for i in range(bb):
        zi = z[i * cin * kh:(i + 1) * cin * kh, :].reshape(cin, kh, 6 * wop)
        zc = zc_ref[i * cin:(i + 1) * cin, :].reshape(cin, 1, w3)
        dy0 = jnp.concatenate([zc, zi[:, 0:kh - 1, w3:2 * w3]], axis=1)
        zc_ref[i * cin:(i + 1) * cin, :] = zi[:, kh - 1, w3:2 * w3]
        pieces = []
        for dy_block in (dy0, zi[:, :, 0:w3], zi[:, :, w3:2 * w3]):
            for dx in range(3):
                pieces.append(dy_block[:, :, dx * wop:(dx + 1) * wop])
        t = jnp.concatenate(pieces, axis=0)

        conv = jax.lax.dot_general(t, wk, (((0,), (0,)), ((), ())),
                                   preferred_element_type=jnp.float32)

        row_acc = jnp.maximum(conv[0] + bias, 0.0)
        for y in range(1, kh):
            row_acc += jnp.maximum(conv[y] + bias, 0.0)

        sacc = jnp.dot(pmx, row_acc, preferred_element_type=jnp.float32)
        pooled_ref[i, 0] = sacc[:s]
        gacc_ref[i:i + 1, :] += sacc[s:s + 1]

    @pl.when(oy == pl.num_programs(1) - 1)
    def _():
        g = gacc_ref[...] * inv_count
        a = jnp.dot(g, wf_ref[...], preferred_element_type=jnp.float32)
        attrs_ref[...] = (a + bf_ref[...]).reshape(bb, 1, -1)


@jax.jit
def _forward(images, conv_w, conv_b, fc_w, fc_b):
    b, cin, h, w = images.shape
    cenc = conv_w.shape[0]
    na = fc_w.shape[0]
    s = 14
    ho = (h - 1) // 2 + 1
    wo = (w - 1) // 2 + 1
    kh, kw = ho // s, wo // s
    wop = _rup(wo, 128)
    cpad = _rup(cenc, 128)
    apad = _rup(na, 128)
    bb = _BB
    while b % bb:
        bb //= 2

    xp = images.reshape(b, cin, h // 2, 2 * w)

    wcol = jnp.arange(w)[:, None]
    xs = jnp.arange(wop)[None, :]
    qhalf = [((wcol == 2 * xs + dx - 1) & (xs < wo)) for dx in range(3)]
    zblk = jnp.zeros((w, wop), jnp.bool_)
    q = jnp.block([[qhalf[0], qhalf[1], qhalf[2], zblk, zblk, zblk],
                   [zblk, zblk, zblk, qhalf[0], qhalf[1], qhalf[2]]])
    q = q.astype(jnp.bfloat16)

    wt = jnp.transpose(conv_w, (2, 3, 1, 0))
    wt = jnp.pad(wt, ((0, 0), (0, 0), (0, 0), (0, cpad - cenc)))
    wk = wt.reshape(9 * cin, cpad).astype(jnp.bfloat16)
    bc = jnp.pad(conv_b, (0, cpad - cenc)).reshape(1, cpad).astype(jnp.float32)

    xcol = jnp.arange(wop)[None, :]
    pm = (xcol // kw == jnp.arange(s)[:, None]) & (xcol < wo)
    pm = pm.astype(jnp.float32) / float(kh * kw)
    pm = jnp.concatenate([pm, (xcol < wo).astype(jnp.float32)], axis=0)

    wf = jnp.pad(jnp.transpose(fc_w),
                 ((0, cpad - cenc), (0, apad - na))).astype(jnp.float32)
    bf2 = jnp.pad(fc_b, (0, apad - na)).reshape(1, apad).astype(jnp.float32)

    cost = pl.CostEstimate(
        flops=(2 * b * ho * wo * 9 * cin * cpad
               + 2 * b * s * cin * kh * 2 * w * 6 * wop
               + 2 * b * s * (s + 1) * wop * cpad),
        transcendentals=0,
        bytes_accessed=(b * cin * h * w * 4
                        + b * s * s * cpad * 4 + b * apad * 4),
    )

    pooled, attrs = pl.pallas_call(
        functools.partial(_enc_kernel, kh=kh, wop=wop, cin=cin, bb=bb,
                          inv_count=1.0 / float(ho * wo)),
        out_shape=(
            jax.ShapeDtypeStruct((b, s, s, cpad), jnp.float32),
            jax.ShapeDtypeStruct((b, 1, apad), jnp.float32),
        ),
        grid=(b // bb, s),
        in_specs=[
            pl.BlockSpec((bb, cin, kh, 2 * w), lambda bb_, oy: (bb_, 0, oy, 0)),
            pl.BlockSpec((2 * w, 6 * wop), lambda bb_, oy: (0, 0)),
            pl.BlockSpec((9 * cin, cpad), lambda bb_, oy: (0, 0)),
            pl.BlockSpec((1, cpad), lambda bb_, oy: (0, 0)),
            pl.BlockSpec((s + 1, wop), lambda bb_, oy: (0, 0)),
            pl.BlockSpec((cpad, apad), lambda bb_, oy: (0, 0)),
            pl.BlockSpec((1, apad), lambda bb_, oy: (0, 0)),
        ],
        out_specs=(
            pl.BlockSpec((bb, 1, s, cpad), lambda bb_, oy: (bb_, oy, 0, 0)),
            pl.BlockSpec((bb, 1, apad), lambda bb_, oy: (bb_, 0, 0)),
        ),
        scratch_shapes=[pltpu.VMEM((bb, cpad), jnp.float32),
                        pltpu.VMEM((bb * cin, 3 * wop), jnp.bfloat16)],
        compiler_params=pltpu.CompilerParams(
            dimension_semantics=("parallel", "arbitrary"),
            vmem_limit_bytes=_VMEM_LIMIT),
        cost_estimate=cost,
    )(xp, q, wk, bc, pm, wf, bf2)

    return pooled[:, :, :, :cenc], attrs[:, 0, :na]


def kernel(images, conv_w, conv_b, fc_w, fc_b):
    return _forward(images, conv_w, conv_b, fc_w, fc_b)


# speedup vs baseline: 2.5976x; 1.0934x over previous
import functools

import jax
import jax.numpy as jnp
from jax.experimental import pallas as pl
from jax.experimental.pallas import tpu as pltpu

_VMEM_LIMIT = 32 * 1024 * 1024
_BB = 8


def _rup(x, m):
    return ((x + m - 1) // m) * m


def _enc_kernel(xm_ref, q_ref, wk_ref, bc_ref, pm_ref, wf_ref,
                bf_ref, pooled_ref, attrs_ref, gacc_ref, zc_ref,
                *, kh, wop, cin, bb, inv_count):
    oy = pl.program_id(1)
    s = pooled_ref.shape[2]
    cpad = pooled_ref.shape[3]
    w3 = 3 * wop

    @pl.when(oy == 0)
    def _():
        gacc_ref[...] = jnp.zeros_like(gacc_ref)
        zc_ref[...] = jnp.zeros_like(zc_ref)

    x2 = xm_ref[...].reshape(bb * cin * kh, -1).astype(jnp.bfloat16)

    z = jnp.dot(x2, q_ref[...], preferred_element_type=jnp.float32)
    z = z.astype(jnp.bfloat16)

    bias = bc_ref[...]
    pmx = pm_ref[...]
    wk = wk_ref[...]
    for i in range(bb):
        zi = z[i * cin * kh:(i + 1) * cin * kh, :].reshape(cin, kh, 6 * wop)
        zc = zc_ref[i * cin:(i + 1) * cin, :].reshape(cin, 1, w3)
        dy0 = jnp.concatenate([zc, zi[:, 0:kh - 1, w3:2 * w3]], axis=1)
        zc_ref[i * cin:(i + 1) * cin, :] = zi[:, kh - 1, w3:2 * w3]
        pieces = []
        for dy_block in (dy0, zi[:, :, 0:w3], zi[:, :, w3:2 * w3]):
            for dx in range(3):
                pieces.append(dy_block[:, :, dx * wop:(dx + 1) * wop])
        t = jnp.concatenate(pieces, axis=0)

        conv = jax.lax.dot_general(t, wk, (((0,), (0,)), ((), ())),
                                   preferred_element_type=jnp.float32)

        row_acc = jnp.maximum(conv[0] + bias, 0.0)
        for y in range(1, kh):
            row_acc += jnp.maximum(conv[y] + bias, 0.0)

        sacc = jnp.dot(pmx, row_acc, preferred_element_type=jnp.float32)
        pooled_ref[i, 0] = sacc[:s]
        gacc_ref[i:i + 1, :] += sacc[s:s + 1]

    @pl.when(oy == pl.num_programs(1) - 1)
    def _():
        g = gacc_ref[...] * inv_count
        a = jnp.dot(g, wf_ref[...], preferred_element_type=jnp.float32)
        attrs_ref[...] = (a + bf_ref[...]).reshape(bb, 1, -1)


@jax.jit
def _forward(images, conv_w, conv_b, fc_w, fc_b):
    b, cin, h, w = images.shape
    cenc = conv_w.shape[0]
    na = fc_w.shape[0]
    s = 14
    ho = (h - 1) // 2 + 1
    wo = (w - 1) // 2 + 1
    kh, kw = ho // s, wo // s
    wop = _rup(wo, 128)
    cpad = _rup(cenc, 128)
    apad = _rup(na, 128)
    bb = _BB
    while b % bb:
        bb //= 2

    xp = images.reshape(b, cin, h // 2, 2 * w)

    wcol = jnp.arange(w)[:, None]
    xs = jnp.arange(wop)[None, :]
    qhalf = [((wcol == 2 * xs + dx - 1) & (xs < wo)) for dx in range(3)]
    zblk = jnp.zeros((w, wop), jnp.bool_)
    q = jnp.block([[qhalf[0], qhalf[1], qhalf[2], zblk, zblk, zblk],
                   [zblk, zblk, zblk, qhalf[0], qhalf[1], qhalf[2]]])
    q = q.astype(jnp.bfloat16)

    wt = jnp.transpose(conv_w, (2, 3, 1, 0))
    wt = jnp.pad(wt, ((0, 0), (0, 0), (0, 0), (0, cpad - cenc)))
    wk = wt.reshape(9 * cin, cpad).astype(jnp.bfloat16)
    bc = jnp.pad(conv_b, (0, cpad - cenc)).reshape(1, cpad).astype(jnp.float32)

    xcol = jnp.arange(wop)[None, :]
    pm = (xcol // kw == jnp.arange(s)[:, None]) & (xcol < wo)
    pm = pm.astype(jnp.float32) / float(kh * kw)
    pm = jnp.concatenate([pm, (xcol < wo).astype(jnp.float32)], axis=0)

    wf = jnp.pad(jnp.transpose(fc_w),
                 ((0, cpad - cenc), (0, apad - na))).astype(jnp.float32)
    bf2 = jnp.pad(fc_b, (0, apad - na)).reshape(1, apad).astype(jnp.float32)

    cost = pl.CostEstimate(
        flops=(2 * b * ho * wo * 9 * cin * cpad
               + 2 * b * s * cin * kh * 2 * w * 6 * wop
               + 2 * b * s * (s + 1) * wop * cpad),
        transcendentals=0,
        bytes_accessed=(b * cin * h * w * 4
                        + b * s * s * cpad * 4 + b * apad * 4),
    )

    pooled, attrs = pl.pallas_call(
        functools.partial(_enc_kernel, kh=kh, wop=wop, cin=cin, bb=bb,
                          inv_count=1.0 / float(ho * wo)),
        out_shape=(
            jax.ShapeDtypeStruct((b, s, s, cpad), jnp.float32),
            jax.ShapeDtypeStruct((b, 1, apad), jnp.float32),
        ),
        grid=(b // bb, s),
        in_specs=[
            pl.BlockSpec((bb, cin, kh, 2 * w), lambda bb_, oy: (bb_, 0, oy, 0)),
            pl.BlockSpec((2 * w, 6 * wop), lambda bb_, oy: (0, 0)),
            pl.BlockSpec((9 * cin, cpad), lambda bb_, oy: (0, 0)),
            pl.BlockSpec((1, cpad), lambda bb_, oy: (0, 0)),
            pl.BlockSpec((s + 1, wop), lambda bb_, oy: (0, 0)),
            pl.BlockSpec((cpad, apad), lambda bb_, oy: (0, 0)),
            pl.BlockSpec((1, apad), lambda bb_, oy: (0, 0)),
        ],
        out_specs=(
            pl.BlockSpec((bb, 1, s, cpad), lambda bb_, oy: (bb_, oy, 0, 0)),
            pl.BlockSpec((bb, 1, apad), lambda bb_, oy: (bb_, 0, 0)),
        ),
        scratch_shapes=[pltpu.VMEM((bb, cpad), jnp.float32),
                        pltpu.VMEM((bb * cin, 3 * wop), jnp.bfloat16)],
        compiler_params=pltpu.CompilerParams(
            dimension_semantics=("parallel", "arbitrary"),
            vmem_limit_bytes=_VMEM_LIMIT),
        cost_estimate=cost,
    )(xp, q, wk, bc, pm, wf, bf2)

    return pooled[:, :, :, :cenc], attrs[:, 0, :na]


def kernel(images, conv_w, conv_b, fc_w, fc_b):
    return _forward(images, conv_w, conv_b, fc_w, fc_b)


# speedup vs baseline: 2.7425x; 1.0558x over previous
import functools

import jax
import jax.numpy as jnp
from jax.experimental import pallas as pl
from jax.experimental.pallas import tpu as pltpu

_VMEM_LIMIT = 32 * 1024 * 1024
_BB = 16


def _rup(x, m):
    return ((x + m - 1) // m) * m


def _enc_kernel(xm_ref, q_ref, wk_ref, bc_ref, pm_ref, wf_ref,
                bf_ref, pooled_ref, attrs_ref, gacc_ref, zc_ref,
                *, kh, wop, cin, bb, inv_count):
    oy = pl.program_id(1)
    s = pooled_ref.shape[2]
    cpad = pooled_ref.shape[3]
    w3 = 3 * wop

    @pl.when(oy == 0)
    def _():
        gacc_ref[...] = jnp.zeros_like(gacc_ref)
        zc_ref[...] = jnp.zeros_like(zc_ref)

    x2 = xm_ref[...].reshape(bb * cin * kh, -1).astype(jnp.bfloat16)

    z = jnp.dot(x2, q_ref[...], preferred_element_type=jnp.float32)
    z = z.astype(jnp.bfloat16)

    bias = bc_ref[...]
    pmx = pm_ref[...]
    wk = wk_ref[...]
    for i in range(bb):
        zi = z[i * cin * kh:(i + 1) * cin * kh, :].reshape(cin, kh, 6 * wop)
        zc = zc_ref[i * cin:(i + 1) * cin, :].reshape(cin, 1, w3)
        dy0 = jnp.concatenate([zc, zi[:, 0:kh - 1, w3:2 * w3]], axis=1)
        zc_ref[i * cin:(i + 1) * cin, :] = zi[:, kh - 1, w3:2 * w3]
        pieces = []
        for dy_block in (dy0, zi[:, :, 0:w3], zi[:, :, w3:2 * w3]):
            for dx in range(3):
                pieces.append(dy_block[:, :, dx * wop:(dx + 1) * wop])
        t = jnp.concatenate(pieces, axis=0)

        conv = jax.lax.dot_general(t, wk, (((0,), (0,)), ((), ())),
                                   preferred_element_type=jnp.float32)

        row_acc = jnp.maximum(conv[0] + bias, 0.0)
        for y in range(1, kh):
            row_acc += jnp.maximum(conv[y] + bias, 0.0)

        sacc = jnp.dot(pmx, row_acc, preferred_element_type=jnp.float32)
        pooled_ref[i, 0] = sacc[:s]
        gacc_ref[i:i + 1, :] += sacc[s:s + 1]

    @pl.when(oy == pl.num_programs(1) - 1)
    def _():
        g = gacc_ref[...] * inv_count
        a = jnp.dot(g, wf_ref[...], preferred_element_type=jnp.float32)
        attrs_ref[...] = (a + bf_ref[...]).reshape(bb, 1, -1)


@jax.jit
def _forward(images, conv_w, conv_b, fc_w, fc_b):
    b, cin, h, w = images.shape
    cenc = conv_w.shape[0]
    na = fc_w.shape[0]
    s = 14
    ho = (h - 1) // 2 + 1
    wo = (w - 1) // 2 + 1
    kh, kw = ho // s, wo // s
    wop = _rup(wo, 128)
    cpad = _rup(cenc, 128)
    apad = _rup(na, 128)
    bb = _BB
    while b % bb:
        bb //= 2

    xp = images.reshape(b, cin, h // 2, 2 * w)

    wcol = jnp.arange(w)[:, None]
    xs = jnp.arange(wop)[None, :]
    qhalf = [((wcol == 2 * xs + dx - 1) & (xs < wo)) for dx in range(3)]
    zblk = jnp.zeros((w, wop), jnp.bool_)
    q = jnp.block([[qhalf[0], qhalf[1], qhalf[2], zblk, zblk, zblk],
                   [zblk, zblk, zblk, qhalf[0], qhalf[1], qhalf[2]]])
    q = q.astype(jnp.bfloat16)

    wt = jnp.transpose(conv_w, (2, 3, 1, 0))
    wt = jnp.pad(wt, ((0, 0), (0, 0), (0, 0), (0, cpad - cenc)))
    wk = wt.reshape(9 * cin, cpad).astype(jnp.bfloat16)
    bc = jnp.pad(conv_b, (0, cpad - cenc)).reshape(1, cpad).astype(jnp.float32)

    xcol = jnp.arange(wop)[None, :]
    pm = (xcol // kw == jnp.arange(s)[:, None]) & (xcol < wo)
    pm = pm.astype(jnp.float32) / float(kh * kw)
    pm = jnp.concatenate([pm, (xcol < wo).astype(jnp.float32)], axis=0)

    wf = jnp.pad(jnp.transpose(fc_w),
                 ((0, cpad - cenc), (0, apad - na))).astype(jnp.float32)
    bf2 = jnp.pad(fc_b, (0, apad - na)).reshape(1, apad).astype(jnp.float32)

    cost = pl.CostEstimate(
        flops=(2 * b * ho * wo * 9 * cin * cpad
               + 2 * b * s * cin * kh * 2 * w * 6 * wop
               + 2 * b * s * (s + 1) * wop * cpad),
        transcendentals=0,
        bytes_accessed=(b * cin * h * w * 4
                        + b * s * s * cpad * 4 + b * apad * 4),
    )

    pooled, attrs = pl.pallas_call(
        functools.partial(_enc_kernel, kh=kh, wop=wop, cin=cin, bb=bb,
                          inv_count=1.0 / float(ho * wo)),
        out_shape=(
            jax.ShapeDtypeStruct((b, s, s, cpad), jnp.float32),
            jax.ShapeDtypeStruct((b, 1, apad), jnp.float32),
        ),
        grid=(b // bb, s),
        in_specs=[
            pl.BlockSpec((bb, cin, kh, 2 * w), lambda bb_, oy: (bb_, 0, oy, 0)),
            pl.BlockSpec((2 * w, 6 * wop), lambda bb_, oy: (0, 0)),
            pl.BlockSpec((9 * cin, cpad), lambda bb_, oy: (0, 0)),
            pl.BlockSpec((1, cpad), lambda bb_, oy: (0, 0)),
            pl.BlockSpec((s + 1, wop), lambda bb_, oy: (0, 0)),
            pl.BlockSpec((cpad, apad), lambda bb_, oy: (0, 0)),
            pl.BlockSpec((1, apad), lambda bb_, oy: (0, 0)),
        ],
        out_specs=(
            pl.BlockSpec((bb, 1, s, cpad), lambda bb_, oy: (bb_, oy, 0, 0)),
            pl.BlockSpec((bb, 1, apad), lambda bb_, oy: (bb_, 0, 0)),
        ),
        scratch_shapes=[pltpu.VMEM((bb, cpad), jnp.float32),
                        pltpu.VMEM((bb * cin, 3 * wop), jnp.bfloat16)],
        compiler_params=pltpu.CompilerParams(
            dimension_semantics=("parallel", "arbitrary"),
            vmem_limit_bytes=_VMEM_LIMIT),
        cost_estimate=cost,
    )(xp, q, wk, bc, pm, wf, bf2)

    return pooled[:, :, :, :cenc], attrs[:, 0, :na]


def kernel(images, conv_w, conv_b, fc_w, fc_b):
    return _forward(images, conv_w, conv_b, fc_w, fc_b)


# speedup vs baseline: 2.7909x; 1.0177x over previous
import functools

import jax
import jax.numpy as jnp
from jax.experimental import pallas as pl
from jax.experimental.pallas import tpu as pltpu

_VMEM_LIMIT = 32 * 1024 * 1024
_BB = 32


def _rup(x, m):
    return ((x + m - 1) // m) * m


def _enc_kernel(xm_ref, q_ref, wk_ref, bc_ref, pm_ref, wf_ref,
                bf_ref, pooled_ref, attrs_ref, gacc_ref, zc_ref,
                *, kh, wop, cin, bb, inv_count):
    oy = pl.program_id(1)
    s = pooled_ref.shape[2]
    cpad = pooled_ref.shape[3]
    w3 = 3 * wop

    @pl.when(oy == 0)
    def _():
        gacc_ref[...] = jnp.zeros_like(gacc_ref)
        zc_ref[...] = jnp.zeros_like(zc_ref)

    x2 = xm_ref[...].reshape(bb * cin * kh, -1).astype(jnp.bfloat16)

    z = jnp.dot(x2, q_ref[...], preferred_element_type=jnp.float32)
    z = z.astype(jnp.bfloat16)

    bias = bc_ref[...]
    pmx = pm_ref[...]
    wk = wk_ref[...]
    for i in range(bb):
        zi = z[i * cin * kh:(i + 1) * cin * kh, :].reshape(cin, kh, 6 * wop)
        zc = zc_ref[i * cin:(i + 1) * cin, :].reshape(cin, 1, w3)
        dy0 = jnp.concatenate([zc, zi[:, 0:kh - 1, w3:2 * w3]], axis=1)
        zc_ref[i * cin:(i + 1) * cin, :] = zi[:, kh - 1, w3:2 * w3]
        pieces = []
        for dy_block in (dy0, zi[:, :, 0:w3], zi[:, :, w3:2 * w3]):
            for dx in range(3):
                pieces.append(dy_block[:, :, dx * wop:(dx + 1) * wop])
        t = jnp.concatenate(pieces, axis=0)

        conv = jax.lax.dot_general(t, wk, (((0,), (0,)), ((), ())),
                                   preferred_element_type=jnp.float32)

        row_acc = jnp.maximum(conv[0] + bias, 0.0)
        for y in range(1, kh):
            row_acc += jnp.maximum(conv[y] + bias, 0.0)

        sacc = jnp.dot(pmx, row_acc, preferred_element_type=jnp.float32)
        pooled_ref[i, 0] = sacc[:s]
        gacc_ref[i:i + 1, :] += sacc[s:s + 1]

    @pl.when(oy == pl.num_programs(1) - 1)
    def _():
        g = gacc_ref[...] * inv_count
        a = jnp.dot(g, wf_ref[...], preferred_element_type=jnp.float32)
        attrs_ref[...] = (a + bf_ref[...]).reshape(bb, 1, -1)


@jax.jit
def _forward(images, conv_w, conv_b, fc_w, fc_b):
    b, cin, h, w = images.shape
    cenc = conv_w.shape[0]
    na = fc_w.shape[0]
    s = 14
    ho = (h - 1) // 2 + 1
    wo = (w - 1) // 2 + 1
    kh, kw = ho // s, wo // s
    wop = _rup(wo, 128)
    cpad = _rup(cenc, 128)
    apad = _rup(na, 128)
    bb = _BB
    while b % bb:
        bb //= 2

    xp = images.reshape(b, cin, h // 2, 2 * w)

    wcol = jnp.arange(w)[:, None]
    xs = jnp.arange(wop)[None, :]
    qhalf = [((wcol == 2 * xs + dx - 1) & (xs < wo)) for dx in range(3)]
    zblk = jnp.zeros((w, wop), jnp.bool_)
    q = jnp.block([[qhalf[0], qhalf[1], qhalf[2], zblk, zblk, zblk],
                   [zblk, zblk, zblk, qhalf[0], qhalf[1], qhalf[2]]])
    q = q.astype(jnp.bfloat16)

    wt = jnp.transpose(conv_w, (2, 3, 1, 0))
    wt = jnp.pad(wt, ((0, 0), (0, 0), (0, 0), (0, cpad - cenc)))
    wk = wt.reshape(9 * cin, cpad).astype(jnp.bfloat16)
    bc = jnp.pad(conv_b, (0, cpad - cenc)).reshape(1, cpad).astype(jnp.float32)

    xcol = jnp.arange(wop)[None, :]
    pm = (xcol // kw == jnp.arange(s)[:, None]) & (xcol < wo)
    pm = pm.astype(jnp.float32) / float(kh * kw)
    pm = jnp.concatenate([pm, (xcol < wo).astype(jnp.float32)], axis=0)

    wf = jnp.pad(jnp.transpose(fc_w),
                 ((0, cpad - cenc), (0, apad - na))).astype(jnp.float32)
    bf2 = jnp.pad(fc_b, (0, apad - na)).reshape(1, apad).astype(jnp.float32)

    cost = pl.CostEstimate(
        flops=(2 * b * ho * wo * 9 * cin * cpad
               + 2 * b * s * cin * kh * 2 * w * 6 * wop
               + 2 * b * s * (s + 1) * wop * cpad),
        transcendentals=0,
        bytes_accessed=(b * cin * h * w * 4
                        + b * s * s * cpad * 4 + b * apad * 4),
    )

    pooled, attrs = pl.pallas_call(
        functools.partial(_enc_kernel, kh=kh, wop=wop, cin=cin, bb=bb,
                          inv_count=1.0 / float(ho * wo)),
        out_shape=(
            jax.ShapeDtypeStruct((b, s, s, cpad), jnp.float32),
            jax.ShapeDtypeStruct((b, 1, apad), jnp.float32),
        ),
        grid=(b // bb, s),
        in_specs=[
            pl.BlockSpec((bb, cin, kh, 2 * w), lambda bb_, oy: (bb_, 0, oy, 0)),
            pl.BlockSpec((2 * w, 6 * wop), lambda bb_, oy: (0, 0)),
            pl.BlockSpec((9 * cin, cpad), lambda bb_, oy: (0, 0)),
            pl.BlockSpec((1, cpad), lambda bb_, oy: (0, 0)),
            pl.BlockSpec((s + 1, wop), lambda bb_, oy: (0, 0)),
            pl.BlockSpec((cpad, apad), lambda bb_, oy: (0, 0)),
            pl.BlockSpec((1, apad), lambda bb_, oy: (0, 0)),
        ],
        out_specs=(
            pl.BlockSpec((bb, 1, s, cpad), lambda bb_, oy: (bb_, oy, 0, 0)),
            pl.BlockSpec((bb, 1, apad), lambda bb_, oy: (bb_, 0, 0)),
        ),
        scratch_shapes=[pltpu.VMEM((bb, cpad), jnp.float32),
                        pltpu.VMEM((bb * cin, 3 * wop), jnp.bfloat16)],
        compiler_params=pltpu.CompilerParams(
            dimension_semantics=("parallel", "arbitrary"),
            vmem_limit_bytes=_VMEM_LIMIT),
        cost_estimate=cost,
    )(xp, q, wk, bc, pm, wf, bf2)

    return pooled[:, :, :, :cenc], attrs[:, 0, :na]


def kernel(images, conv_w, conv_b, fc_w, fc_b):
    return _forward(images, conv_w, conv_b, fc_w, fc_b)


# speedup vs baseline: 3.1980x; 1.1459x over previous
import functools

import jax
import jax.numpy as jnp
from jax.experimental import pallas as pl
from jax.experimental.pallas import tpu as pltpu

_VMEM_LIMIT = 32 * 1024 * 1024
_BB = 16


def _rup(x, m):
    return ((x + m - 1) // m) * m


def _enc_kernel(xm_ref, q_ref, wk_ref, bc_ref, pm_ref, wf_ref,
                bf_ref, pooled_ref, attrs_ref, gacc_ref, zc_ref,
                *, kh, wop, cin, bb, sh, inv_count):
    oy = pl.program_id(1)
    s = pooled_ref.shape[2]
    cpad = pooled_ref.shape[3]
    w3 = 3 * wop
    nr = sh * kh

    @pl.when(oy == 0)
    def _():
        gacc_ref[...] = jnp.zeros_like(gacc_ref)
        zc_ref[...] = jnp.zeros_like(zc_ref)

    x2 = xm_ref[...].reshape(bb * cin * nr, -1).astype(jnp.bfloat16)

    z = jnp.dot(x2, q_ref[...], preferred_element_type=jnp.float32)
    z = z.astype(jnp.bfloat16)

    bias = bc_ref[...]
    pmx = pm_ref[...]
    wk = wk_ref[...]
    for i in range(bb):
        zi = z[i * cin * nr:(i + 1) * cin * nr, :].reshape(cin, nr, 6 * wop)
        zc = zc_ref[i * cin:(i + 1) * cin, :].reshape(cin, 1, w3)
        dy0 = jnp.concatenate([zc, zi[:, 0:nr - 1, w3:2 * w3]], axis=1)
        zc_ref[i * cin:(i + 1) * cin, :] = zi[:, nr - 1, w3:2 * w3]
        pieces = []
        for dy_block in (dy0, zi[:, :, 0:w3], zi[:, :, w3:2 * w3]):
            for dx in range(3):
                pieces.append(dy_block[:, :, dx * wop:(dx + 1) * wop])
        t = jnp.concatenate(pieces, axis=0)

        conv = jax.lax.dot_general(t, wk, (((0,), (0,)), ((), ())),
                                   preferred_element_type=jnp.float32)

        for r in range(sh):
            row_acc = jnp.maximum(conv[r * kh] + bias, 0.0)
            for y in range(r * kh + 1, (r + 1) * kh):
                row_acc += jnp.maximum(conv[y] + bias, 0.0)

            sacc = jnp.dot(pmx, row_acc, preferred_element_type=jnp.float32)
            pooled_ref[i, r] = sacc[:s]
            gacc_ref[i:i + 1, :] += sacc[s:s + 1]

    @pl.when(oy == pl.num_programs(1) - 1)
    def _():
        g = gacc_ref[...] * inv_count
        a = jnp.dot(g, wf_ref[...], preferred_element_type=jnp.float32)
        attrs_ref[...] = (a + bf_ref[...]).reshape(bb, 1, -1)


@jax.jit
def _forward(images, conv_w, conv_b, fc_w, fc_b):
    b, cin, h, w = images.shape
    cenc = conv_w.shape[0]
    na = fc_w.shape[0]
    s = 14
    ho = (h - 1) // 2 + 1
    wo = (w - 1) // 2 + 1
    kh, kw = ho // s, wo // s
    wop = _rup(wo, 128)
    cpad = _rup(cenc, 128)
    apad = _rup(na, 128)
    bb = _BB
    while b % bb:
        bb //= 2
    sh = 2 if s % 2 == 0 else 1

    xp = images.reshape(b, cin, h // 2, 2 * w)

    wcol = jnp.arange(w)[:, None]
    xs = jnp.arange(wop)[None, :]
    qhalf = [((wcol == 2 * xs + dx - 1) & (xs < wo)) for dx in range(3)]
    zblk = jnp.zeros((w, wop), jnp.bool_)
    q = jnp.block([[qhalf[0], qhalf[1], qhalf[2], zblk, zblk, zblk],
                   [zblk, zblk, zblk, qhalf[0], qhalf[1], qhalf[2]]])
    q = q.astype(jnp.bfloat16)

    wt = jnp.transpose(conv_w, (2, 3, 1, 0))
    wt = jnp.pad(wt, ((0, 0), (0, 0), (0, 0), (0, cpad - cenc)))
    wk = wt.reshape(9 * cin, cpad).astype(jnp.bfloat16)
    bc = jnp.pad(conv_b, (0, cpad - cenc)).reshape(1, cpad).astype(jnp.float32)

    xcol = jnp.arange(wop)[None, :]
    pm = (xcol // kw == jnp.arange(s)[:, None]) & (xcol < wo)
    pm = pm.astype(jnp.float32) / float(kh * kw)
    pm = jnp.concatenate([pm, (xcol < wo).astype(jnp.float32)], axis=0)

    wf = jnp.pad(jnp.transpose(fc_w),
                 ((0, cpad - cenc), (0, apad - na))).astype(jnp.float32)
    bf2 = jnp.pad(fc_b, (0, apad - na)).reshape(1, apad).astype(jnp.float32)

    cost = pl.CostEstimate(
        flops=(2 * b * ho * wo * 9 * cin * cpad
               + 2 * b * s * cin * kh * 2 * w * 6 * wop
               + 2 * b * s * (s + 1) * wop * cpad),
        transcendentals=0,
        bytes_accessed=(b * cin * h * w * 4
                        + b * s * s * cpad * 4 + b * apad * 4),
    )

    pooled, attrs = pl.pallas_call(
        functools.partial(_enc_kernel, kh=kh, wop=wop, cin=cin, bb=bb,
                          sh=sh, inv_count=1.0 / float(ho * wo)),
        out_shape=(
            jax.ShapeDtypeStruct((b, s, s, cpad), jnp.float32),
            jax.ShapeDtypeStruct((b, 1, apad), jnp.float32),
        ),
        grid=(b // bb, s // sh),
        in_specs=[
            pl.BlockSpec((bb, cin, sh * kh, 2 * w),
                         lambda bb_, oy: (bb_, 0, oy, 0)),
            pl.BlockSpec((2 * w, 6 * wop), lambda bb_, oy: (0, 0)),
            pl.BlockSpec((9 * cin, cpad), lambda bb_, oy: (0, 0)),
            pl.BlockSpec((1, cpad), lambda bb_, oy: (0, 0)),
            pl.BlockSpec((s + 1, wop), lambda bb_, oy: (0, 0)),
            pl.BlockSpec((cpad, apad), lambda bb_, oy: (0, 0)),
            pl.BlockSpec((1, apad), lambda bb_, oy: (0, 0)),
        ],
        out_specs=(
            pl.BlockSpec((bb, sh, s, cpad), lambda bb_, oy: (bb_, oy, 0, 0)),
            pl.BlockSpec((bb, 1, apad), lambda bb_, oy: (bb_, 0, 0)),
        ),
        scratch_shapes=[pltpu.VMEM((bb, cpad), jnp.float32),
                        pltpu.VMEM((bb * cin, 3 * wop), jnp.bfloat16)],
        compiler_params=pltpu.CompilerParams(
            dimension_semantics=("parallel", "arbitrary"),
            vmem_limit_bytes=_VMEM_LIMIT),
        cost_estimate=cost,
    )(xp, q, wk, bc, pm, wf, bf2)

    return pooled[:, :, :, :cenc], attrs[:, 0, :na]


def kernel(images, conv_w, conv_b, fc_w, fc_b):
    return _forward(images, conv_w, conv_b, fc_w, fc_b)
